# Initial kernel scaffold; baseline (speedup 1.0000x reference)
import functools

import jax
import jax.numpy as jnp
from jax import lax
from jax.experimental import pallas as pl
from jax.experimental.pallas import tpu as pltpu
from jax.experimental.pallas import tpu_sc as plsc

N_USERS = 50000
N_ITEMS = 50000
NN = N_USERS + N_ITEMS
D = 16
K_LAYERS = 3
NE = 3200000
BATCH = 4096

NC = 2
NS = 16
NW = NC * NS
LANES = 16

SUB = 128
BLK = 1024
NB = 98
NE_PAD = NW * BLK * NB

ROWS_SC = NN // NS
ZCH = 625
ROWS_W = NN // NW

_MESH = plsc.VectorSubcoreMesh(core_axis_name="c", subcore_axis_name="s")


@functools.partial(
    pl.kernel,
    out_type=jax.ShapeDtypeStruct((NC * NN, D), jnp.float32),
    mesh=_MESH,
    scratch_types=[
        pltpu.VMEM_SHARED((NN, D), jnp.float32),
        pltpu.VMEM((BLK // SUB, SUB), jnp.int32),
        pltpu.VMEM((BLK // SUB, SUB), jnp.int32),
        pltpu.VMEM((BLK,), jnp.float32),
        pltpu.VMEM((BLK, D), jnp.float32),
        pltpu.VMEM((ZCH, D), jnp.float32),
        pltpu.SemaphoreType.DMA,
    ],
)
def _layer(e_hbm, src_hbm, dst_hbm, w_hbm, out_hbm,
           acc, srcv, dstv, wv, rows, zbuf, gsem):
    cid = lax.axis_index("c")
    sid = lax.axis_index("s")
    wid = cid * NS + sid

    @pl.loop(0, ZCH)
    def _zb(r):
        zbuf[r] = jnp.zeros((D,), jnp.float32)

    @pl.loop(0, ROWS_SC // ZCH)
    def _zacc(k):
        pltpu.sync_copy(zbuf, acc.at[pl.ds(sid * ROWS_SC + k * ZCH, ZCH)])

    plsc.subcore_barrier()

    @pl.loop(0, NB)
    def _block(b):
        base_row = (wid * NB + b) * (BLK // SUB)
        base_edge = (wid * NB + b) * BLK
        pltpu.sync_copy(src_hbm.at[pl.ds(base_row, BLK // SUB)], srcv)
        pltpu.sync_copy(dst_hbm.at[pl.ds(base_row, BLK // SUB)], dstv)
        pltpu.sync_copy(w_hbm.at[pl.ds(base_edge, BLK)], wv)

        descs = [
            pltpu.async_copy(e_hbm.at[srcv.at[j]],
                             rows.at[pl.ds(j * SUB, SUB)], gsem)
            for j in range(BLK // SUB)
        ]
        for d in descs:
            d.wait()

        @pl.loop(0, BLK, unroll=8)
        def _scale(i):
            wb = plsc.load_gather(wv, [jnp.full((LANES,), i, jnp.int32)])
            rows[i] = rows[i] * wb

        for j in range(BLK // SUB):
            pltpu.sync_copy(rows.at[pl.ds(j * SUB, SUB)],
                            acc.at[dstv.at[j]], add=True)

    plsc.subcore_barrier()

    @pl.loop(0, ROWS_SC // ZCH)
    def _wb(k):
        r0 = sid * ROWS_SC + k * ZCH
        pltpu.sync_copy(acc.at[pl.ds(r0, ZCH)],
                        out_hbm.at[pl.ds(cid * NN + r0, ZCH)])


def _combine_body(last, a_hbm, s_hbm, *refs):
    if last:
        ef_hbm, b0, b1, bs = refs
    else:
        e_hbm, sn_hbm, b0, b1, bs = refs
    wid = lax.axis_index("c") * NS + lax.axis_index("s")

    @pl.loop(0, ROWS_W // ZCH)
    def _chunk(k):
        r0 = wid * ROWS_W + k * ZCH
        pltpu.sync_copy(a_hbm.at[pl.ds(r0, ZCH)], b0)
        pltpu.sync_copy(a_hbm.at[pl.ds(NN + r0, ZCH)], b1)
        pltpu.sync_copy(s_hbm.at[pl.ds(r0, ZCH)], bs)

        @pl.loop(0, ZCH, unroll=4)
        def _row(r):
            e = b0[r] + b1[r]
            if last:
                b0[r] = (bs[r] + e) * jnp.float32(1.0 / (K_LAYERS + 1))
            else:
                b0[r] = e
                bs[r] = bs[r] + e

        if last:
            pltpu.sync_copy(b0, ef_hbm.at[pl.ds(r0, ZCH)])
        else:
            pltpu.sync_copy(b0, e_hbm.at[pl.ds(r0, ZCH)])
            pltpu.sync_copy(bs, sn_hbm.at[pl.ds(r0, ZCH)])


_COMBINE_SCRATCH = [
    pltpu.VMEM((ZCH, D), jnp.float32),
    pltpu.VMEM((ZCH, D), jnp.float32),
    pltpu.VMEM((ZCH, D), jnp.float32),
]

_combine_mid = functools.partial(
    pl.kernel,
    out_type=(jax.ShapeDtypeStruct((NN, D), jnp.float32),
              jax.ShapeDtypeStruct((NN, D), jnp.float32)),
    mesh=_MESH,
    scratch_types=_COMBINE_SCRATCH,
)(functools.partial(_combine_body, False))

_combine_last = functools.partial(
    pl.kernel,
    out_type=jax.ShapeDtypeStruct((NN, D), jnp.float32),
    mesh=_MESH,
    scratch_types=_COMBINE_SCRATCH,
)(functools.partial(_combine_body, True))


N_IDX = 3 * BATCH
G_PER_W = N_IDX // NW


@functools.partial(
    pl.kernel,
    out_type=jax.ShapeDtypeStruct((N_IDX, D), jnp.float32),
    mesh=_MESH,
    scratch_types=[
        pltpu.VMEM((G_PER_W // SUB, SUB), jnp.int32),
        pltpu.VMEM((G_PER_W, D), jnp.float32),
        pltpu.SemaphoreType.DMA,
    ],
)
def _batch_gather(e_hbm, idx_hbm, out_hbm, iv, rbuf, sem):
    wid = lax.axis_index("c") * NS + lax.axis_index("s")
    nrow = G_PER_W // SUB
    pltpu.sync_copy(idx_hbm.at[pl.ds(wid * nrow, nrow)], iv)
    descs = [
        pltpu.async_copy(e_hbm.at[iv.at[j]],
                         rbuf.at[pl.ds(j * SUB, SUB)], sem)
        for j in range(nrow)
    ]
    for d in descs:
        d.wait()
    pltpu.sync_copy(rbuf, out_hbm.at[pl.ds(wid * G_PER_W, G_PER_W)])


def kernel(users, pos_items, neg_items, user_emb, item_emb,
           edge_src, edge_dst, edge_w):
    E0 = jnp.concatenate([user_emb, item_emb], axis=0)
    pad = NE_PAD - NE
    src2 = jnp.pad(edge_src, (0, pad)).reshape(-1, SUB)
    dst2 = jnp.pad(edge_dst, (0, pad)).reshape(-1, SUB)
    w1 = jnp.pad(edge_w, (0, pad))

    E, S = E0, E0
    for k in range(K_LAYERS):
        A = _layer(E, src2, dst2, w1)
        if k < K_LAYERS - 1:
            E, S = _combine_mid(A, S)
        else:
            E_final = _combine_last(A, S)

    idx = jnp.concatenate(
        [users, N_USERS + pos_items, N_USERS + neg_items]).reshape(-1, SUB)
    rows = _batch_gather(E_final, idx)
    return (rows[:BATCH], rows[BATCH:2 * BATCH], rows[2 * BATCH:])


# speedup vs baseline: 29.1366x; 29.1366x over previous
import functools

import jax
import jax.numpy as jnp
from jax import lax
from jax.experimental import pallas as pl
from jax.experimental.pallas import tpu as pltpu
from jax.experimental.pallas import tpu_sc as plsc

N_USERS = 50000
N_ITEMS = 50000
NN = N_USERS + N_ITEMS
D = 16
K_LAYERS = 3
NE = 3200000
BATCH = 4096

NC = 2
NS = 16
NW = NC * NS
LANES = 16

SUB = 128
BLK = 1024
NB = 98
NE_PAD = NW * BLK * NB

NNP = 100352
ROWS_SC = NNP // NS
ZCH = 784
ZB = 392
ROWS_W = NNP // NW


def _al8(x):
    return pl.multiple_of(x, 8)


_MESH = plsc.VectorSubcoreMesh(core_axis_name="c", subcore_axis_name="s")


@functools.partial(
    pl.kernel,
    out_type=jax.ShapeDtypeStruct((NC * NNP, D), jnp.float32),
    mesh=_MESH,
    compiler_params=pltpu.CompilerParams(use_tc_tiling_on_sc=False),
    scratch_types=[
        pltpu.VMEM_SHARED((NNP, D), jnp.float32),
        pltpu.VMEM((BLK // SUB, SUB), jnp.int32),
        pltpu.VMEM((BLK // SUB, SUB), jnp.int32),
        pltpu.VMEM((BLK,), jnp.float32),
        pltpu.VMEM((BLK, D), jnp.float32),
        pltpu.VMEM((ZB, D), jnp.float32),
        pltpu.SemaphoreType.DMA,
    ],
)
def _layer(e_hbm, src_hbm, dst_hbm, w_hbm, out_hbm,
           acc, srcv, dstv, wv, rows, zbuf, gsem):
    cid = lax.axis_index("c")
    sid = lax.axis_index("s")
    wid = cid * NS + sid

    @pl.loop(0, ZB)
    def _zb(r):
        zbuf[r] = jnp.zeros((D,), jnp.float32)

    @pl.loop(0, ROWS_SC // ZB)
    def _zacc(k):
        pltpu.sync_copy(zbuf, acc.at[pl.ds(_al8(sid * ROWS_SC + k * ZB), ZB)])

    plsc.subcore_barrier()

    @pl.loop(0, NB)
    def _block(b):
        base_row = _al8((wid * NB + b) * (BLK // SUB))
        base_edge = _al8((wid * NB + b) * BLK)
        pltpu.sync_copy(src_hbm.at[pl.ds(base_row, BLK // SUB)], srcv)
        pltpu.sync_copy(dst_hbm.at[pl.ds(base_row, BLK // SUB)], dstv)
        pltpu.sync_copy(w_hbm.at[pl.ds(base_edge, BLK)], wv)

        descs = [
            pltpu.async_copy(e_hbm.at[srcv.at[j]],
                             rows.at[pl.ds(j * SUB, SUB)], gsem)
            for j in range(BLK // SUB)
        ]
        for d in descs:
            d.wait()

        @pl.loop(0, BLK // LANES)
        def _scale(m):
            base = pl.multiple_of(m * LANES, LANES)
            wvec = wv[pl.ds(base, LANES)]
            for j in range(LANES):
                rows[base + j] = rows[base + j] * wvec[j]

        for j in range(BLK // SUB):
            pltpu.sync_copy(rows.at[pl.ds(j * SUB, SUB)],
                            acc.at[dstv.at[j]], add=True)

    plsc.subcore_barrier()

    @pl.loop(0, ROWS_SC // ZCH)
    def _wb(k):
        r0 = _al8(sid * ROWS_SC + k * ZCH)
        pltpu.sync_copy(acc.at[pl.ds(r0, ZCH)],
                        out_hbm.at[pl.ds(_al8(cid * NNP + r0), ZCH)])


def _combine_body(last, a_hbm, s_hbm, *refs):
    if last:
        ef_hbm, b0, b1, bs = refs
    else:
        e_hbm, sn_hbm, b0, b1, bs = refs
    wid = lax.axis_index("c") * NS + lax.axis_index("s")

    @pl.loop(0, ROWS_W // ZCH)
    def _chunk(k):
        r0 = _al8(wid * ROWS_W + k * ZCH)
        pltpu.sync_copy(a_hbm.at[pl.ds(r0, ZCH)], b0)
        pltpu.sync_copy(a_hbm.at[pl.ds(_al8(NNP + r0), ZCH)], b1)
        pltpu.sync_copy(s_hbm.at[pl.ds(r0, ZCH)], bs)

        @pl.loop(0, ZCH, unroll=4)
        def _row(r):
            e = b0[r] + b1[r]
            if last:
                b0[r] = (bs[r] + e) * jnp.float32(1.0 / (K_LAYERS + 1))
            else:
                b0[r] = e
                bs[r] = bs[r] + e

        if last:
            pltpu.sync_copy(b0, ef_hbm.at[pl.ds(r0, ZCH)])
        else:
            pltpu.sync_copy(b0, e_hbm.at[pl.ds(r0, ZCH)])
            pltpu.sync_copy(bs, sn_hbm.at[pl.ds(r0, ZCH)])


_COMBINE_SCRATCH = [
    pltpu.VMEM((ZCH, D), jnp.float32),
    pltpu.VMEM((ZCH, D), jnp.float32),
    pltpu.VMEM((ZCH, D), jnp.float32),
]

_combine_mid = functools.partial(
    pl.kernel,
    out_type=(jax.ShapeDtypeStruct((NNP, D), jnp.float32),
              jax.ShapeDtypeStruct((NNP, D), jnp.float32)),
    mesh=_MESH,
    compiler_params=pltpu.CompilerParams(use_tc_tiling_on_sc=False),
    scratch_types=_COMBINE_SCRATCH,
)(functools.partial(_combine_body, False))

_combine_last = functools.partial(
    pl.kernel,
    out_type=jax.ShapeDtypeStruct((NNP, D), jnp.float32),
    mesh=_MESH,
    compiler_params=pltpu.CompilerParams(use_tc_tiling_on_sc=False),
    scratch_types=_COMBINE_SCRATCH,
)(functools.partial(_combine_body, True))


N_IDX = 3 * BATCH
G_PER_W = N_IDX // NW


@functools.partial(
    pl.kernel,
    out_type=jax.ShapeDtypeStruct((N_IDX, D), jnp.float32),
    mesh=_MESH,
    compiler_params=pltpu.CompilerParams(use_tc_tiling_on_sc=False),
    scratch_types=[
        pltpu.VMEM((G_PER_W,), jnp.int32),
        pltpu.VMEM((G_PER_W, D), jnp.float32),
        pltpu.SemaphoreType.DMA,
    ],
)
def _batch_gather(e_hbm, idx_hbm, out_hbm, iv, rbuf, sem):
    wid = lax.axis_index("c") * NS + lax.axis_index("s")
    pltpu.sync_copy(idx_hbm.at[pl.ds(_al8(wid * G_PER_W), G_PER_W)], iv)
    descs = [
        pltpu.async_copy(e_hbm.at[iv.at[pl.ds(j * SUB, SUB)]],
                         rbuf.at[pl.ds(j * SUB, SUB)], sem)
        for j in range(G_PER_W // SUB)
    ]
    for d in descs:
        d.wait()
    pltpu.sync_copy(rbuf, out_hbm.at[pl.ds(_al8(wid * G_PER_W), G_PER_W)])


def kernel(users, pos_items, neg_items, user_emb, item_emb,
           edge_src, edge_dst, edge_w):
    E0 = jnp.concatenate([user_emb, item_emb], axis=0)
    E0 = jnp.pad(E0, ((0, NNP - NN), (0, 0)))
    pad = NE_PAD - NE
    src2 = jnp.pad(edge_src, (0, pad)).reshape(-1, SUB)
    dst2 = jnp.pad(edge_dst, (0, pad)).reshape(-1, SUB)
    w1 = jnp.pad(edge_w, (0, pad))

    E, S = E0, E0
    for k in range(K_LAYERS):
        A = _layer(E, src2, dst2, w1)
        if k < K_LAYERS - 1:
            E, S = _combine_mid(A, S)
        else:
            E_final = _combine_last(A, S)

    idx = jnp.concatenate([users, N_USERS + pos_items, N_USERS + neg_items])
    rows = _batch_gather(E_final, idx)
    return (rows[:BATCH], rows[BATCH:2 * BATCH], rows[2 * BATCH:])


# speedup vs baseline: 34.5715x; 1.1865x over previous
import functools

import jax
import jax.numpy as jnp
from jax import lax
from jax.experimental import pallas as pl
from jax.experimental.pallas import tpu as pltpu
from jax.experimental.pallas import tpu_sc as plsc

N_USERS = 50000
N_ITEMS = 50000
NN = N_USERS + N_ITEMS
D = 16
K_LAYERS = 3
NE = 3200000
BATCH = 4096

NC = 2
NS = 16
NW = NC * NS
LANES = 16

SUB = 128
BLK = 384
NSUB = BLK // SUB
NB = 262
NE_PAD = NW * BLK * NB

NNP = 100352
ROWS_SC = NNP // NS
ROWS_W = NNP // NW
CCH = 784


def _al8(x):
    return pl.multiple_of(x, 8)


_MESH = plsc.VectorSubcoreMesh(core_axis_name="c", subcore_axis_name="s")
_PARAMS = pltpu.CompilerParams(use_tc_tiling_on_sc=False)


@functools.partial(
    pl.kernel,
    out_type=jax.ShapeDtypeStruct((NC * NNP, D), jnp.float32),
    mesh=_MESH,
    compiler_params=_PARAMS,
    scratch_types=[
        pltpu.VMEM_SHARED((NNP, D), jnp.float32),
        pltpu.VMEM((2, NSUB, SUB), jnp.int32),
        pltpu.VMEM((2, NSUB, SUB), jnp.int32),
        pltpu.VMEM((2, NSUB, SUB), jnp.int32),
        pltpu.VMEM((2, BLK), jnp.float32),
        pltpu.VMEM((2, BLK, D), jnp.float32),
        pltpu.VMEM((2, BLK, D), jnp.float32),
        pltpu.SemaphoreType.DMA,
        pltpu.SemaphoreType.DMA,
        pltpu.SemaphoreType.DMA,
        pltpu.SemaphoreType.DMA,
        pltpu.SemaphoreType.DMA,
        pltpu.SemaphoreType.DMA,
    ],
)
def _layer(e_hbm, z_hbm, src_hbm, dst_hbm, w_hbm, out_hbm,
           acc, srcv, dstv, sdst, wv, rows, msg,
           si0, si1, sg0, sg1, ss0, ss1):
    cid = lax.axis_index("c")
    sid = lax.axis_index("s")
    wid = cid * NS + sid
    si = (si0, si1)
    sg = (sg0, sg1)
    ss = (ss0, ss1)

    r0 = _al8(sid * ROWS_SC)
    pltpu.sync_copy(z_hbm.at[pl.ds(r0, ROWS_SC)], acc.at[pl.ds(r0, ROWS_SC)])
    plsc.subcore_barrier()

    def fire_idx(b, p):
        base_row = (wid * NB + b) * NSUB
        base_edge = _al8((wid * NB + b) * BLK)
        pltpu.async_copy(src_hbm.at[pl.ds(base_row, NSUB)], srcv.at[p], si[p])
        pltpu.async_copy(dst_hbm.at[pl.ds(base_row, NSUB)], dstv.at[p], si[p])
        pltpu.async_copy(w_hbm.at[pl.ds(base_edge, BLK)], wv.at[p], si[p])

    def wait_idx(p):
        pltpu.make_async_copy(src_hbm.at[pl.ds(0, NSUB)], srcv.at[p],
                              si[p]).wait()
        pltpu.make_async_copy(dst_hbm.at[pl.ds(0, NSUB)], dstv.at[p],
                              si[p]).wait()
        pltpu.make_async_copy(w_hbm.at[pl.ds(0, BLK)], wv.at[p], si[p]).wait()

    def fire_gather(p):
        for j in range(NSUB):
            pltpu.async_copy(e_hbm.at[srcv.at[p, j]],
                             rows.at[p, pl.ds(j * SUB, SUB)], sg[p])

    def wait_gather(p):
        for j in range(NSUB):
            pltpu.make_async_copy(e_hbm.at[srcv.at[p, j]],
                                  rows.at[p, pl.ds(j * SUB, SUB)],
                                  sg[p]).wait()

    def fire_scatter(p):
        for j in range(NSUB):
            pltpu.async_copy(msg.at[p, pl.ds(j * SUB, SUB)],
                             acc.at[sdst.at[p, j]], ss[p], add=True)

    def wait_scatter(p):
        for j in range(NSUB):
            pltpu.make_async_copy(msg.at[p, pl.ds(j * SUB, SUB)],
                                  acc.at[sdst.at[p, j]], ss[p]).wait()

    fire_idx(0, 0)
    wait_idx(0)
    fire_gather(0)
    fire_idx(1, 1)

    def half_iter(bb, p):
        b = bb * 2 + p

        wait_gather(p)

        @pl.when(b >= 2)
        def _():
            wait_scatter(p)

        for j in range(NSUB):
            for m in range(SUB // LANES):
                sdst[p, j, pl.ds(m * LANES, LANES)] = \
                    dstv[p, j, pl.ds(m * LANES, LANES)]

        @pl.loop(0, BLK // LANES)
        def _scale(mi):
            base = pl.multiple_of(mi * LANES, LANES)
            wvec = wv[p, pl.ds(base, LANES)]
            for j in range(LANES):
                msg[p, base + j] = rows[p, base + j] * wvec[j]

        fire_scatter(p)

        @pl.when(b + 2 < NB)
        def _():
            fire_idx(b + 2, p)

        @pl.when(b + 1 < NB)
        def _():
            wait_idx(1 - p)
            fire_gather(1 - p)

    @pl.loop(0, NB // 2)
    def _bb(bb):
        half_iter(bb, 0)
        half_iter(bb, 1)

    wait_scatter(0)
    wait_scatter(1)

    plsc.subcore_barrier()
    pltpu.sync_copy(acc.at[pl.ds(r0, ROWS_SC)],
                    out_hbm.at[pl.ds(_al8(cid * NNP + r0), ROWS_SC)])


def _combine_body(last, a_hbm, s_hbm, *refs):
    if last:
        ef_hbm, b0, b1, bs, sem = refs
    else:
        e_hbm, sn_hbm, b0, b1, bs, sem = refs
    wid = lax.axis_index("c") * NS + lax.axis_index("s")

    @pl.loop(0, ROWS_W // CCH)
    def _chunk(k):
        r0 = _al8(wid * ROWS_W + k * CCH)
        d0 = pltpu.async_copy(a_hbm.at[pl.ds(r0, CCH)], b0, sem)
        d1 = pltpu.async_copy(a_hbm.at[pl.ds(_al8(NNP + r0), CCH)], b1, sem)
        d2 = pltpu.async_copy(s_hbm.at[pl.ds(r0, CCH)], bs, sem)
        d0.wait()
        d1.wait()
        d2.wait()

        @pl.loop(0, CCH, unroll=8)
        def _row(r):
            e = b0[r] + b1[r]
            if last:
                b0[r] = (bs[r] + e) * jnp.float32(1.0 / (K_LAYERS + 1))
            else:
                b0[r] = e
                bs[r] = bs[r] + e

        if last:
            pltpu.sync_copy(b0, ef_hbm.at[pl.ds(r0, CCH)])
        else:
            d3 = pltpu.async_copy(b0, e_hbm.at[pl.ds(r0, CCH)], sem)
            d4 = pltpu.async_copy(bs, sn_hbm.at[pl.ds(r0, CCH)], sem)
            d3.wait()
            d4.wait()


_COMBINE_SCRATCH = [
    pltpu.VMEM((CCH, D), jnp.float32),
    pltpu.VMEM((CCH, D), jnp.float32),
    pltpu.VMEM((CCH, D), jnp.float32),
    pltpu.SemaphoreType.DMA,
]

_combine_mid = functools.partial(
    pl.kernel,
    out_type=(jax.ShapeDtypeStruct((NNP, D), jnp.float32),
              jax.ShapeDtypeStruct((NNP, D), jnp.float32)),
    mesh=_MESH,
    compiler_params=_PARAMS,
    scratch_types=_COMBINE_SCRATCH,
)(functools.partial(_combine_body, False))

_combine_last = functools.partial(
    pl.kernel,
    out_type=jax.ShapeDtypeStruct((NNP, D), jnp.float32),
    mesh=_MESH,
    compiler_params=_PARAMS,
    scratch_types=_COMBINE_SCRATCH,
)(functools.partial(_combine_body, True))


N_IDX = 3 * BATCH
G_PER_W = N_IDX // NW


@functools.partial(
    pl.kernel,
    out_type=jax.ShapeDtypeStruct((N_IDX, D), jnp.float32),
    mesh=_MESH,
    compiler_params=_PARAMS,
    scratch_types=[
        pltpu.VMEM((G_PER_W,), jnp.int32),
        pltpu.VMEM((G_PER_W, D), jnp.float32),
        pltpu.SemaphoreType.DMA,
    ],
)
def _batch_gather(e_hbm, idx_hbm, out_hbm, iv, rbuf, sem):
    wid = lax.axis_index("c") * NS + lax.axis_index("s")
    pltpu.sync_copy(idx_hbm.at[pl.ds(_al8(wid * G_PER_W), G_PER_W)], iv)
    descs = [
        pltpu.async_copy(e_hbm.at[iv.at[pl.ds(j * SUB, SUB)]],
                         rbuf.at[pl.ds(j * SUB, SUB)], sem)
        for j in range(G_PER_W // SUB)
    ]
    for d in descs:
        d.wait()
    pltpu.sync_copy(rbuf, out_hbm.at[pl.ds(_al8(wid * G_PER_W), G_PER_W)])


def kernel(users, pos_items, neg_items, user_emb, item_emb,
           edge_src, edge_dst, edge_w):
    E0 = jnp.concatenate([user_emb, item_emb], axis=0)
    E0 = jnp.pad(E0, ((0, NNP - NN), (0, 0)))
    Z = jnp.zeros((NNP, D), jnp.float32)
    pad = NE_PAD - NE
    src2 = jnp.pad(edge_src, (0, pad)).reshape(-1, SUB)
    dst2 = jnp.pad(edge_dst, (0, pad)).reshape(-1, SUB)
    w1 = jnp.pad(edge_w, (0, pad))

    E, S = E0, E0
    for k in range(K_LAYERS):
        A = _layer(E, Z, src2, dst2, w1)
        if k < K_LAYERS - 1:
            E, S = _combine_mid(A, S)
        else:
            E_final = _combine_last(A, S)

    idx = jnp.concatenate([users, N_USERS + pos_items, N_USERS + neg_items])
    rows = _batch_gather(E_final, idx)
    return (rows[:BATCH], rows[BATCH:2 * BATCH], rows[2 * BATCH:])
